# Initial kernel scaffold; baseline (speedup 1.0000x reference)
import jax
import jax.numpy as jnp
from jax.experimental import pallas as pl


def kernel(xyz1, xyz2, points1, points2, W0, b0, g0, be0, W1, b1, g1, be1):
    raise NotImplementedError("write your pallas kernel here")


# speedup vs baseline: 23.5361x; 23.5361x over previous
import jax
import jax.numpy as jnp
from jax.experimental import pallas as pl

_EPS = 1e-5


def _pass1_kernel(x1_ref, x2t_ref, p1_ref, p2_ref, w0t_ref, b0_ref,
                  y1_ref, ssum_ref, ssq_ref):
    b = pl.program_id(0)
    i = pl.program_id(1)

    x1 = x1_ref[0]
    x2t = x2t_ref[0]
    dots = jnp.dot(x1, x2t, preferred_element_type=jnp.float32)
    x1sq = jnp.sum(x1 * x1, axis=1, keepdims=True)
    x2sq = jnp.sum(x2t * x2t, axis=0, keepdims=True)
    d = x1sq + x2sq - 2.0 * dots

    nb, m = d.shape
    iota = jax.lax.broadcasted_iota(jnp.int32, (nb, m), 1)

    vals = []
    idxs = []
    dw = d
    for _ in range(3):
        v = jnp.min(dw, axis=1, keepdims=True)
        idx = jnp.min(jnp.where(dw == v, iota, m), axis=1, keepdims=True)
        vals.append(v)
        idxs.append(idx)
        dw = jnp.where(iota == idx, jnp.float32(jnp.inf), dw)

    ws = [1.0 / jnp.maximum(v, 1e-10) for v in vals]
    wsum = ws[0] + ws[1] + ws[2]

    wmat = jnp.where(iota == idxs[0], ws[0] / wsum, 0.0)
    wmat = wmat + jnp.where(iota == idxs[1], ws[1] / wsum, 0.0)
    wmat = wmat + jnp.where(iota == idxs[2], ws[2] / wsum, 0.0)

    interp = jnp.dot(wmat, p2_ref[0], preferred_element_type=jnp.float32)
    x = jnp.concatenate([p1_ref[0], interp], axis=1)
    y1 = jnp.dot(x, w0t_ref[...], preferred_element_type=jnp.float32)
    y1 = y1 + b0_ref[...]
    y1_ref[0] = y1

    @pl.when(jnp.logical_and(b == 0, i == 0))
    def _():
        ssum_ref[...] = jnp.zeros_like(ssum_ref)
        ssq_ref[...] = jnp.zeros_like(ssq_ref)

    ssum_ref[...] += jnp.sum(y1, axis=0, keepdims=True)
    ssq_ref[...] += jnp.sum(y1 * y1, axis=0, keepdims=True)


def _pass2_kernel(y1_ref, sc_ref, sh_ref, w1t_ref, b1_ref,
                  y2_ref, ssum_ref, ssq_ref):
    i = pl.program_id(0)
    h = jnp.maximum(y1_ref[...] * sc_ref[...] + sh_ref[...], 0.0)
    y2 = jnp.dot(h, w1t_ref[...], preferred_element_type=jnp.float32)
    y2 = y2 + b1_ref[...]
    y2_ref[...] = y2

    @pl.when(i == 0)
    def _():
        ssum_ref[...] = jnp.zeros_like(ssum_ref)
        ssq_ref[...] = jnp.zeros_like(ssq_ref)

    ssum_ref[...] += jnp.sum(y2, axis=0, keepdims=True)
    ssq_ref[...] += jnp.sum(y2 * y2, axis=0, keepdims=True)


def _pass3_kernel(y2_ref, sc_ref, sh_ref, o_ref):
    o_ref[...] = jnp.maximum(y2_ref[...] * sc_ref[...] + sh_ref[...], 0.0)


def kernel(xyz1, xyz2, points1, points2, W0, b0, g0, be0, W1, b1, g1, be1):
    f32 = jnp.float32
    B, N, _ = xyz1.shape
    M = xyz2.shape[1]
    D1 = points1.shape[2]
    D2 = points2.shape[2]
    O1 = W0.shape[0]
    O2 = W1.shape[0]
    R = B * N

    NB_BLK = 512
    NB = N // NB_BLK

    xyz1p = jnp.pad(xyz1, ((0, 0), (0, 0), (0, 5)))
    xyz2t = jnp.pad(xyz2, ((0, 0), (0, 0), (0, 5))).transpose(0, 2, 1)
    w0t = W0.T
    w1t = W1.T
    b0r = b0[None, :]
    b1r = b1[None, :]

    y1, s1, q1 = pl.pallas_call(
        _pass1_kernel,
        grid=(B, NB),
        in_specs=[
            pl.BlockSpec((1, NB_BLK, 8), lambda b, i: (b, i, 0)),
            pl.BlockSpec((1, 8, M), lambda b, i: (b, 0, 0)),
            pl.BlockSpec((1, NB_BLK, D1), lambda b, i: (b, i, 0)),
            pl.BlockSpec((1, M, D2), lambda b, i: (b, 0, 0)),
            pl.BlockSpec((D1 + D2, O1), lambda b, i: (0, 0)),
            pl.BlockSpec((1, O1), lambda b, i: (0, 0)),
        ],
        out_specs=[
            pl.BlockSpec((1, NB_BLK, O1), lambda b, i: (b, i, 0)),
            pl.BlockSpec((1, O1), lambda b, i: (0, 0)),
            pl.BlockSpec((1, O1), lambda b, i: (0, 0)),
        ],
        out_shape=[
            jax.ShapeDtypeStruct((B, N, O1), f32),
            jax.ShapeDtypeStruct((1, O1), f32),
            jax.ShapeDtypeStruct((1, O1), f32),
        ],
    )(xyz1p, xyz2t, points1, points2, w0t, b0r)

    mean1 = s1[0] / R
    var1 = q1[0] / R - mean1 * mean1
    sc1 = g0 / jnp.sqrt(var1 + _EPS)
    sh1 = be0 - mean1 * sc1

    R_BLK = 2048
    G = R // R_BLK
    y1f = y1.reshape(R, O1)

    y2, s2, q2 = pl.pallas_call(
        _pass2_kernel,
        grid=(G,),
        in_specs=[
            pl.BlockSpec((R_BLK, O1), lambda i: (i, 0)),
            pl.BlockSpec((1, O1), lambda i: (0, 0)),
            pl.BlockSpec((1, O1), lambda i: (0, 0)),
            pl.BlockSpec((O1, O2), lambda i: (0, 0)),
            pl.BlockSpec((1, O2), lambda i: (0, 0)),
        ],
        out_specs=[
            pl.BlockSpec((R_BLK, O2), lambda i: (i, 0)),
            pl.BlockSpec((1, O2), lambda i: (0, 0)),
            pl.BlockSpec((1, O2), lambda i: (0, 0)),
        ],
        out_shape=[
            jax.ShapeDtypeStruct((R, O2), f32),
            jax.ShapeDtypeStruct((1, O2), f32),
            jax.ShapeDtypeStruct((1, O2), f32),
        ],
    )(y1f, sc1[None, :], sh1[None, :], w1t, b1r)

    mean2 = s2[0] / R
    var2 = q2[0] / R - mean2 * mean2
    sc2 = g1 / jnp.sqrt(var2 + _EPS)
    sh2 = be1 - mean2 * sc2

    out = pl.pallas_call(
        _pass3_kernel,
        grid=(G,),
        in_specs=[
            pl.BlockSpec((R_BLK, O2), lambda i: (i, 0)),
            pl.BlockSpec((1, O2), lambda i: (0, 0)),
            pl.BlockSpec((1, O2), lambda i: (0, 0)),
        ],
        out_specs=pl.BlockSpec((R_BLK, O2), lambda i: (i, 0)),
        out_shape=jax.ShapeDtypeStruct((R, O2), f32),
    )(y2, sc2[None, :], sh2[None, :])

    return out.reshape(B, N, O2)


# speedup vs baseline: 26.4883x; 1.1254x over previous
import jax
import jax.numpy as jnp
from jax.experimental import pallas as pl

_EPS = 1e-5


def _pass1_kernel(x1_ref, x2ts_ref, p1_ref, p2_ref, w0t_ref, b0_ref,
                  y1_ref, ssum_ref, ssq_ref):
    b = pl.program_id(0)
    i = pl.program_id(1)

    x1 = x1_ref[0]
    x2ts = x2ts_ref[0]
    dots = jnp.dot(x1, x2ts, preferred_element_type=jnp.float32)
    x2sq = 0.25 * jnp.sum(x2ts * x2ts, axis=0, keepdims=True)
    dsel = dots + x2sq
    x1sq = jnp.sum(x1 * x1, axis=1, keepdims=True)

    nb, m = dsel.shape
    fiota = jax.lax.broadcasted_iota(jnp.int32, (nb, m), 1).astype(jnp.float32)
    mf = jnp.float32(m)

    dw = dsel
    wmat = None
    wsum = None
    for k in range(3):
        v = jnp.min(dw, axis=1, keepdims=True)
        idxf = jnp.min(jnp.where(dw == v, fiota, mf), axis=1, keepdims=True)
        w_k = 1.0 / jnp.maximum(v + x1sq, 1e-10)
        wsum = w_k if k == 0 else wsum + w_k
        m_k = fiota == idxf
        wmat = jnp.where(m_k, w_k, 0.0 if k == 0 else wmat)
        if k < 2:
            dw = jnp.where(m_k, jnp.float32(jnp.inf), dw)

    interp = jnp.dot(wmat, p2_ref[0], preferred_element_type=jnp.float32)
    interp = interp * (1.0 / wsum)
    x = jnp.concatenate([p1_ref[0], interp], axis=1)
    y1 = jnp.dot(x, w0t_ref[...], preferred_element_type=jnp.float32)
    y1 = y1 + b0_ref[...]
    y1_ref[0] = y1

    @pl.when(jnp.logical_and(b == 0, i == 0))
    def _():
        ssum_ref[...] = jnp.zeros_like(ssum_ref)
        ssq_ref[...] = jnp.zeros_like(ssq_ref)

    ssum_ref[...] += jnp.sum(y1, axis=0, keepdims=True)
    ssq_ref[...] += jnp.sum(y1 * y1, axis=0, keepdims=True)


def _pass2_kernel(y1_ref, sc_ref, sh_ref, w1t_ref, b1_ref,
                  y2_ref, ssum_ref, ssq_ref):
    i = pl.program_id(0)
    h = jnp.maximum(y1_ref[...] * sc_ref[...] + sh_ref[...], 0.0)
    y2 = jnp.dot(h, w1t_ref[...], preferred_element_type=jnp.float32)
    y2 = y2 + b1_ref[...]
    y2_ref[...] = y2

    @pl.when(i == 0)
    def _():
        ssum_ref[...] = jnp.zeros_like(ssum_ref)
        ssq_ref[...] = jnp.zeros_like(ssq_ref)

    ssum_ref[...] += jnp.sum(y2, axis=0, keepdims=True)
    ssq_ref[...] += jnp.sum(y2 * y2, axis=0, keepdims=True)


def _pass3_kernel(y2_ref, sc_ref, sh_ref, o_ref):
    o_ref[...] = jnp.maximum(y2_ref[...] * sc_ref[...] + sh_ref[...], 0.0)


def kernel(xyz1, xyz2, points1, points2, W0, b0, g0, be0, W1, b1, g1, be1):
    f32 = jnp.float32
    B, N, _ = xyz1.shape
    M = xyz2.shape[1]
    D1 = points1.shape[2]
    D2 = points2.shape[2]
    O1 = W0.shape[0]
    O2 = W1.shape[0]
    R = B * N

    NB_BLK = 512
    NB = N // NB_BLK

    xyz1p = jnp.pad(xyz1, ((0, 0), (0, 0), (0, 5)))
    xyz2t = (-2.0 * jnp.pad(xyz2, ((0, 0), (0, 0), (0, 5)))).transpose(0, 2, 1)
    w0t = W0.T
    w1t = W1.T
    b0r = b0[None, :]
    b1r = b1[None, :]

    y1, s1, q1 = pl.pallas_call(
        _pass1_kernel,
        grid=(B, NB),
        in_specs=[
            pl.BlockSpec((1, NB_BLK, 8), lambda b, i: (b, i, 0)),
            pl.BlockSpec((1, 8, M), lambda b, i: (b, 0, 0)),
            pl.BlockSpec((1, NB_BLK, D1), lambda b, i: (b, i, 0)),
            pl.BlockSpec((1, M, D2), lambda b, i: (b, 0, 0)),
            pl.BlockSpec((D1 + D2, O1), lambda b, i: (0, 0)),
            pl.BlockSpec((1, O1), lambda b, i: (0, 0)),
        ],
        out_specs=[
            pl.BlockSpec((1, NB_BLK, O1), lambda b, i: (b, i, 0)),
            pl.BlockSpec((1, O1), lambda b, i: (0, 0)),
            pl.BlockSpec((1, O1), lambda b, i: (0, 0)),
        ],
        out_shape=[
            jax.ShapeDtypeStruct((B, N, O1), f32),
            jax.ShapeDtypeStruct((1, O1), f32),
            jax.ShapeDtypeStruct((1, O1), f32),
        ],
    )(xyz1p, xyz2t, points1, points2, w0t, b0r)

    mean1 = s1[0] / R
    var1 = q1[0] / R - mean1 * mean1
    sc1 = g0 / jnp.sqrt(var1 + _EPS)
    sh1 = be0 - mean1 * sc1

    R_BLK = 2048
    G = R // R_BLK
    y1f = y1.reshape(R, O1)

    y2, s2, q2 = pl.pallas_call(
        _pass2_kernel,
        grid=(G,),
        in_specs=[
            pl.BlockSpec((R_BLK, O1), lambda i: (i, 0)),
            pl.BlockSpec((1, O1), lambda i: (0, 0)),
            pl.BlockSpec((1, O1), lambda i: (0, 0)),
            pl.BlockSpec((O1, O2), lambda i: (0, 0)),
            pl.BlockSpec((1, O2), lambda i: (0, 0)),
        ],
        out_specs=[
            pl.BlockSpec((R_BLK, O2), lambda i: (i, 0)),
            pl.BlockSpec((1, O2), lambda i: (0, 0)),
            pl.BlockSpec((1, O2), lambda i: (0, 0)),
        ],
        out_shape=[
            jax.ShapeDtypeStruct((R, O2), f32),
            jax.ShapeDtypeStruct((1, O2), f32),
            jax.ShapeDtypeStruct((1, O2), f32),
        ],
    )(y1f, sc1[None, :], sh1[None, :], w1t, b1r)

    mean2 = s2[0] / R
    var2 = q2[0] / R - mean2 * mean2
    sc2 = g1 / jnp.sqrt(var2 + _EPS)
    sh2 = be1 - mean2 * sc2

    out = pl.pallas_call(
        _pass3_kernel,
        grid=(G,),
        in_specs=[
            pl.BlockSpec((R_BLK, O2), lambda i: (i, 0)),
            pl.BlockSpec((1, O2), lambda i: (0, 0)),
            pl.BlockSpec((1, O2), lambda i: (0, 0)),
        ],
        out_specs=pl.BlockSpec((R_BLK, O2), lambda i: (i, 0)),
        out_shape=jax.ShapeDtypeStruct((R, O2), f32),
    )(y2, sc2[None, :], sh2[None, :])

    return out.reshape(B, N, O2)


# speedup vs baseline: 30.6865x; 1.1585x over previous
import jax
import jax.numpy as jnp
from jax.experimental import pallas as pl

_EPS = 1e-5


def _pass1_kernel(x1_ref, x2ts_ref, p1_ref, p2_ref, w0t_ref, b0_ref,
                  y1_ref, ssum_ref, ssq_ref):
    b = pl.program_id(0)
    i = pl.program_id(1)

    x1 = x1_ref[0]
    x2ts = x2ts_ref[0]
    dots = jnp.dot(x1, x2ts, preferred_element_type=jnp.float32)
    x2sq = 0.25 * jnp.sum(x2ts * x2ts, axis=0, keepdims=True)
    dsel = dots + x2sq
    x1sq = jnp.sum(x1 * x1, axis=1, keepdims=True)

    dw = dsel
    wmat = None
    wsum = None
    for k in range(3):
        v = jnp.min(dw, axis=1, keepdims=True)
        w_k = 1.0 / jnp.maximum(v + x1sq, 1e-10)
        wsum = w_k if k == 0 else wsum + w_k
        m_k = dw == v
        wmat = jnp.where(m_k, w_k, 0.0 if k == 0 else wmat)
        if k < 2:
            dw = jnp.where(m_k, jnp.float32(jnp.inf), dw)

    interp = jnp.dot(wmat, p2_ref[0], preferred_element_type=jnp.float32)
    interp = interp * (1.0 / wsum)
    x = jnp.concatenate([p1_ref[0], interp], axis=1)
    y1 = jnp.dot(x, w0t_ref[...], preferred_element_type=jnp.float32)
    y1 = y1 + b0_ref[...]
    y1_ref[0] = y1

    @pl.when(jnp.logical_and(b == 0, i == 0))
    def _():
        ssum_ref[...] = jnp.zeros_like(ssum_ref)
        ssq_ref[...] = jnp.zeros_like(ssq_ref)

    ssum_ref[...] += jnp.sum(y1, axis=0, keepdims=True)
    ssq_ref[...] += jnp.sum(y1 * y1, axis=0, keepdims=True)


def _pass2_kernel(y1_ref, sc_ref, sh_ref, w1t_ref, b1_ref,
                  y2_ref, ssum_ref, ssq_ref):
    i = pl.program_id(0)
    h = jnp.maximum(y1_ref[...] * sc_ref[...] + sh_ref[...], 0.0)
    y2 = jnp.dot(h, w1t_ref[...], preferred_element_type=jnp.float32)
    y2 = y2 + b1_ref[...]
    y2_ref[...] = y2

    @pl.when(i == 0)
    def _():
        ssum_ref[...] = jnp.zeros_like(ssum_ref)
        ssq_ref[...] = jnp.zeros_like(ssq_ref)

    ssum_ref[...] += jnp.sum(y2, axis=0, keepdims=True)
    ssq_ref[...] += jnp.sum(y2 * y2, axis=0, keepdims=True)


def _pass3_kernel(y2_ref, sc_ref, sh_ref, o_ref):
    o_ref[...] = jnp.maximum(y2_ref[...] * sc_ref[...] + sh_ref[...], 0.0)


def kernel(xyz1, xyz2, points1, points2, W0, b0, g0, be0, W1, b1, g1, be1):
    f32 = jnp.float32
    B, N, _ = xyz1.shape
    M = xyz2.shape[1]
    D1 = points1.shape[2]
    D2 = points2.shape[2]
    O1 = W0.shape[0]
    O2 = W1.shape[0]
    R = B * N

    NB_BLK = 512
    NB = N // NB_BLK

    xyz1p = jnp.pad(xyz1, ((0, 0), (0, 0), (0, 5)))
    xyz2t = (-2.0 * jnp.pad(xyz2, ((0, 0), (0, 0), (0, 5)))).transpose(0, 2, 1)
    w0t = W0.T
    w1t = W1.T
    b0r = b0[None, :]
    b1r = b1[None, :]

    y1, s1, q1 = pl.pallas_call(
        _pass1_kernel,
        grid=(B, NB),
        in_specs=[
            pl.BlockSpec((1, NB_BLK, 8), lambda b, i: (b, i, 0)),
            pl.BlockSpec((1, 8, M), lambda b, i: (b, 0, 0)),
            pl.BlockSpec((1, NB_BLK, D1), lambda b, i: (b, i, 0)),
            pl.BlockSpec((1, M, D2), lambda b, i: (b, 0, 0)),
            pl.BlockSpec((D1 + D2, O1), lambda b, i: (0, 0)),
            pl.BlockSpec((1, O1), lambda b, i: (0, 0)),
        ],
        out_specs=[
            pl.BlockSpec((1, NB_BLK, O1), lambda b, i: (b, i, 0)),
            pl.BlockSpec((1, O1), lambda b, i: (0, 0)),
            pl.BlockSpec((1, O1), lambda b, i: (0, 0)),
        ],
        out_shape=[
            jax.ShapeDtypeStruct((B, N, O1), f32),
            jax.ShapeDtypeStruct((1, O1), f32),
            jax.ShapeDtypeStruct((1, O1), f32),
        ],
    )(xyz1p, xyz2t, points1, points2, w0t, b0r)

    mean1 = s1[0] / R
    var1 = q1[0] / R - mean1 * mean1
    sc1 = g0 / jnp.sqrt(var1 + _EPS)
    sh1 = be0 - mean1 * sc1

    R_BLK = 2048
    G = R // R_BLK
    y1f = y1.reshape(R, O1)

    y2, s2, q2 = pl.pallas_call(
        _pass2_kernel,
        grid=(G,),
        in_specs=[
            pl.BlockSpec((R_BLK, O1), lambda i: (i, 0)),
            pl.BlockSpec((1, O1), lambda i: (0, 0)),
            pl.BlockSpec((1, O1), lambda i: (0, 0)),
            pl.BlockSpec((O1, O2), lambda i: (0, 0)),
            pl.BlockSpec((1, O2), lambda i: (0, 0)),
        ],
        out_specs=[
            pl.BlockSpec((R_BLK, O2), lambda i: (i, 0)),
            pl.BlockSpec((1, O2), lambda i: (0, 0)),
            pl.BlockSpec((1, O2), lambda i: (0, 0)),
        ],
        out_shape=[
            jax.ShapeDtypeStruct((R, O2), f32),
            jax.ShapeDtypeStruct((1, O2), f32),
            jax.ShapeDtypeStruct((1, O2), f32),
        ],
    )(y1f, sc1[None, :], sh1[None, :], w1t, b1r)

    mean2 = s2[0] / R
    var2 = q2[0] / R - mean2 * mean2
    sc2 = g1 / jnp.sqrt(var2 + _EPS)
    sh2 = be1 - mean2 * sc2

    out = pl.pallas_call(
        _pass3_kernel,
        grid=(G,),
        in_specs=[
            pl.BlockSpec((R_BLK, O2), lambda i: (i, 0)),
            pl.BlockSpec((1, O2), lambda i: (0, 0)),
            pl.BlockSpec((1, O2), lambda i: (0, 0)),
        ],
        out_specs=pl.BlockSpec((R_BLK, O2), lambda i: (i, 0)),
        out_shape=jax.ShapeDtypeStruct((R, O2), f32),
    )(y2, sc2[None, :], sh2[None, :])

    return out.reshape(B, N, O2)


# speedup vs baseline: 31.0415x; 1.0116x over previous
import jax
import jax.numpy as jnp
from jax.experimental import pallas as pl
from jax.experimental.pallas import tpu as pltpu

_EPS = 1e-5


def _pass1_kernel(x1_ref, x2ts_ref, p1_ref, p2_ref, w0t_ref, b0_ref,
                  y1_ref, ssum_ref, ssq_ref):
    x1 = x1_ref[0]
    x2ts = x2ts_ref[0]
    dots = jnp.dot(x1, x2ts, preferred_element_type=jnp.float32)
    x2sq = 0.25 * jnp.sum(x2ts * x2ts, axis=0, keepdims=True)
    dsel = dots + x2sq
    x1sq = jnp.sum(x1 * x1, axis=1, keepdims=True)

    dw = dsel
    wmat = None
    wsum = None
    for k in range(3):
        v = jnp.min(dw, axis=1, keepdims=True)
        w_k = 1.0 / jnp.maximum(v + x1sq, 1e-10)
        wsum = w_k if k == 0 else wsum + w_k
        m_k = dw == v
        wmat = jnp.where(m_k, w_k, 0.0 if k == 0 else wmat)
        if k < 2:
            dw = jnp.where(m_k, jnp.float32(jnp.inf), dw)

    interp = jnp.dot(wmat, p2_ref[0], preferred_element_type=jnp.float32)
    interp = interp * (1.0 / wsum)
    x = jnp.concatenate([p1_ref[0], interp], axis=1)
    y1 = jnp.dot(x, w0t_ref[...], preferred_element_type=jnp.float32)
    y1 = y1 + b0_ref[...]
    y1_ref[0] = y1

    ssum_ref[0, 0] = jnp.sum(y1, axis=0, keepdims=True)
    ssq_ref[0, 0] = jnp.sum(y1 * y1, axis=0, keepdims=True)


def _pass2_kernel(y1_ref, sc_ref, sh_ref, w1t_ref, b1_ref,
                  y2_ref, ssum_ref, ssq_ref):
    h = jnp.maximum(y1_ref[...] * sc_ref[...] + sh_ref[...], 0.0)
    y2 = jnp.dot(h, w1t_ref[...], preferred_element_type=jnp.float32)
    y2 = y2 + b1_ref[...]
    y2_ref[...] = y2
    ssum_ref[0] = jnp.sum(y2, axis=0, keepdims=True)
    ssq_ref[0] = jnp.sum(y2 * y2, axis=0, keepdims=True)


def _pass3_kernel(y2_ref, sc_ref, sh_ref, o_ref):
    o_ref[...] = jnp.maximum(y2_ref[...] * sc_ref[...] + sh_ref[...], 0.0)


def kernel(xyz1, xyz2, points1, points2, W0, b0, g0, be0, W1, b1, g1, be1):
    f32 = jnp.float32
    B, N, _ = xyz1.shape
    M = xyz2.shape[1]
    D1 = points1.shape[2]
    D2 = points2.shape[2]
    O1 = W0.shape[0]
    O2 = W1.shape[0]
    R = B * N

    NB_BLK = 512
    NB = N // NB_BLK

    xyz1p = jnp.pad(xyz1, ((0, 0), (0, 0), (0, 5)))
    xyz2t = (-2.0 * jnp.pad(xyz2, ((0, 0), (0, 0), (0, 5)))).transpose(0, 2, 1)
    w0t = W0.T
    w1t = W1.T
    b0r = b0[None, :]
    b1r = b1[None, :]

    y1, s1p, q1p = pl.pallas_call(
        _pass1_kernel,
        grid=(B, NB),
        in_specs=[
            pl.BlockSpec((1, NB_BLK, 8), lambda b, i: (b, i, 0)),
            pl.BlockSpec((1, 8, M), lambda b, i: (b, 0, 0)),
            pl.BlockSpec((1, NB_BLK, D1), lambda b, i: (b, i, 0)),
            pl.BlockSpec((1, M, D2), lambda b, i: (b, 0, 0)),
            pl.BlockSpec((D1 + D2, O1), lambda b, i: (0, 0)),
            pl.BlockSpec((1, O1), lambda b, i: (0, 0)),
        ],
        out_specs=[
            pl.BlockSpec((1, NB_BLK, O1), lambda b, i: (b, i, 0)),
            pl.BlockSpec((1, 1, 1, O1), lambda b, i: (b, i, 0, 0)),
            pl.BlockSpec((1, 1, 1, O1), lambda b, i: (b, i, 0, 0)),
        ],
        out_shape=[
            jax.ShapeDtypeStruct((B, N, O1), f32),
            jax.ShapeDtypeStruct((B, NB, 1, O1), f32),
            jax.ShapeDtypeStruct((B, NB, 1, O1), f32),
        ],
        compiler_params=pltpu.CompilerParams(
            dimension_semantics=("parallel", "parallel")),
    )(xyz1p, xyz2t, points1, points2, w0t, b0r)

    mean1 = jnp.sum(s1p, axis=(0, 1, 2)) / R
    var1 = jnp.sum(q1p, axis=(0, 1, 2)) / R - mean1 * mean1
    sc1 = g0 / jnp.sqrt(var1 + _EPS)
    sh1 = be0 - mean1 * sc1

    R_BLK = 2048
    G = R // R_BLK
    y1f = y1.reshape(R, O1)

    y2, s2p, q2p = pl.pallas_call(
        _pass2_kernel,
        grid=(G,),
        in_specs=[
            pl.BlockSpec((R_BLK, O1), lambda i: (i, 0)),
            pl.BlockSpec((1, O1), lambda i: (0, 0)),
            pl.BlockSpec((1, O1), lambda i: (0, 0)),
            pl.BlockSpec((O1, O2), lambda i: (0, 0)),
            pl.BlockSpec((1, O2), lambda i: (0, 0)),
        ],
        out_specs=[
            pl.BlockSpec((R_BLK, O2), lambda i: (i, 0)),
            pl.BlockSpec((1, 1, O2), lambda i: (i, 0, 0)),
            pl.BlockSpec((1, 1, O2), lambda i: (i, 0, 0)),
        ],
        out_shape=[
            jax.ShapeDtypeStruct((R, O2), f32),
            jax.ShapeDtypeStruct((G, 1, O2), f32),
            jax.ShapeDtypeStruct((G, 1, O2), f32),
        ],
        compiler_params=pltpu.CompilerParams(
            dimension_semantics=("parallel",)),
    )(y1f, sc1[None, :], sh1[None, :], w1t, b1r)

    mean2 = jnp.sum(s2p, axis=(0, 1)) / R
    var2 = jnp.sum(q2p, axis=(0, 1)) / R - mean2 * mean2
    sc2 = g1 / jnp.sqrt(var2 + _EPS)
    sh2 = be1 - mean2 * sc2

    out = pl.pallas_call(
        _pass3_kernel,
        grid=(G,),
        in_specs=[
            pl.BlockSpec((R_BLK, O2), lambda i: (i, 0)),
            pl.BlockSpec((1, O2), lambda i: (0, 0)),
            pl.BlockSpec((1, O2), lambda i: (0, 0)),
        ],
        out_specs=pl.BlockSpec((R_BLK, O2), lambda i: (i, 0)),
        out_shape=jax.ShapeDtypeStruct((R, O2), f32),
        compiler_params=pltpu.CompilerParams(
            dimension_semantics=("parallel",)),
    )(y2, sc2[None, :], sh2[None, :])

    return out.reshape(B, N, O2)


# speedup vs baseline: 35.6259x; 1.1477x over previous
import jax
import jax.numpy as jnp
from jax.experimental import pallas as pl
from jax.experimental.pallas import tpu as pltpu

_EPS = 1e-5
_NB_BLK = 512
_CHUNK = 2048


def _fp_kernel(x1_ref, x2ts_ref, p1_ref, p2_ref, w0t_ref, b0_ref,
               w1t_ref, b1_ref, g0_ref, be0_ref, g1_ref, be1_ref,
               o_ref, y1s_ref, s1_ref, q1_ref):
    t = pl.program_id(0)
    steps = pl.num_programs(0) - 1
    nb = _NB_BLK

    @pl.when(t == 0)
    def _():
        s1_ref[...] = jnp.zeros_like(s1_ref)
        q1_ref[...] = jnp.zeros_like(q1_ref)

    @pl.when(t < steps)
    def _():
        x1 = x1_ref[0]
        x2ts = x2ts_ref[0]
        dots = jnp.dot(x1, x2ts, preferred_element_type=jnp.float32)
        x2sq = 0.25 * jnp.sum(x2ts * x2ts, axis=0, keepdims=True)
        dsel = dots + x2sq
        x1sq = jnp.sum(x1 * x1, axis=1, keepdims=True)

        dw = dsel
        wmat = None
        wsum = None
        for k in range(3):
            v = jnp.min(dw, axis=1, keepdims=True)
            w_k = 1.0 / jnp.maximum(v + x1sq, 1e-10)
            wsum = w_k if k == 0 else wsum + w_k
            m_k = dw == v
            wmat = jnp.where(m_k, w_k, 0.0 if k == 0 else wmat)
            if k < 2:
                dw = jnp.where(m_k, jnp.float32(jnp.inf), dw)

        interp = jnp.dot(wmat, p2_ref[0], preferred_element_type=jnp.float32)
        interp = interp * (1.0 / wsum)
        x = jnp.concatenate([p1_ref[0], interp], axis=1)
        y1 = jnp.dot(x, w0t_ref[...], preferred_element_type=jnp.float32)
        y1 = y1 + b0_ref[...]
        y1s_ref[pl.ds(t * nb, nb), :] = y1
        s1_ref[...] += jnp.sum(y1, axis=0, keepdims=True)
        q1_ref[...] += jnp.sum(y1 * y1, axis=0, keepdims=True)

    @pl.when(t == steps)
    def _():
        rows = steps * nb
        rcount = jnp.float32(rows)
        mean1 = s1_ref[...] / rcount
        var1 = q1_ref[...] / rcount - mean1 * mean1
        sc1 = g0_ref[...] / jnp.sqrt(var1 + _EPS)
        sh1 = be0_ref[...] - mean1 * sc1

        w1t = w1t_ref[...]
        b1 = b1_ref[...]
        n_chunks = rows // _CHUNK
        s2 = None
        q2 = None
        for c in range(n_chunks):
            y1c = y1s_ref[pl.ds(c * _CHUNK, _CHUNK), :]
            h = jnp.maximum(y1c * sc1 + sh1, 0.0)
            y2c = jnp.dot(h, w1t, preferred_element_type=jnp.float32) + b1
            y1s_ref[pl.ds(c * _CHUNK, _CHUNK), :] = y2c
            cs = jnp.sum(y2c, axis=0, keepdims=True)
            cq = jnp.sum(y2c * y2c, axis=0, keepdims=True)
            s2 = cs if c == 0 else s2 + cs
            q2 = cq if c == 0 else q2 + cq

        mean2 = s2 / rcount
        var2 = q2 / rcount - mean2 * mean2
        sc2 = g1_ref[...] / jnp.sqrt(var2 + _EPS)
        sh2 = be1_ref[...] - mean2 * sc2

        n_per_b = o_ref.shape[1]
        for c in range(n_chunks):
            y2c = y1s_ref[pl.ds(c * _CHUNK, _CHUNK), :]
            res = jnp.maximum(y2c * sc2 + sh2, 0.0)
            bb = (c * _CHUNK) // n_per_b
            rr = (c * _CHUNK) % n_per_b
            o_ref[bb, pl.ds(rr, _CHUNK), :] = res


def kernel(xyz1, xyz2, points1, points2, W0, b0, g0, be0, W1, b1, g1, be1):
    f32 = jnp.float32
    B, N, _ = xyz1.shape
    M = xyz2.shape[1]
    D1 = points1.shape[2]
    D2 = points2.shape[2]
    O1 = W0.shape[0]
    O2 = W1.shape[0]
    R = B * N

    nb = _NB_BLK
    NB = N // nb
    steps = B * NB

    xyz1p = jnp.pad(xyz1, ((0, 0), (0, 0), (0, 5)))
    xyz2t = (-2.0 * jnp.pad(xyz2, ((0, 0), (0, 0), (0, 5)))).transpose(0, 2, 1)
    w0t = W0.T
    w1t = W1.T

    def bmap(t):
        tc = jnp.minimum(t, steps - 1)
        return (tc // NB, tc % NB, 0)

    def cmap(t):
        return (jnp.minimum(t, steps - 1) // NB, 0, 0)

    out = pl.pallas_call(
        _fp_kernel,
        grid=(steps + 1,),
        in_specs=[
            pl.BlockSpec((1, nb, 8), bmap),
            pl.BlockSpec((1, 8, M), cmap),
            pl.BlockSpec((1, nb, D1), bmap),
            pl.BlockSpec((1, M, D2), cmap),
            pl.BlockSpec((D1 + D2, O1), lambda t: (0, 0)),
            pl.BlockSpec((1, O1), lambda t: (0, 0)),
            pl.BlockSpec((O1, O2), lambda t: (0, 0)),
            pl.BlockSpec((1, O2), lambda t: (0, 0)),
            pl.BlockSpec((1, O1), lambda t: (0, 0)),
            pl.BlockSpec((1, O1), lambda t: (0, 0)),
            pl.BlockSpec((1, O2), lambda t: (0, 0)),
            pl.BlockSpec((1, O2), lambda t: (0, 0)),
        ],
        out_specs=pl.BlockSpec((B, N, O2), lambda t: (0, 0, 0)),
        out_shape=jax.ShapeDtypeStruct((B, N, O2), f32),
        scratch_shapes=[
            pltpu.VMEM((R, O1), f32),
            pltpu.VMEM((1, O1), f32),
            pltpu.VMEM((1, O1), f32),
        ],
        compiler_params=pltpu.CompilerParams(
            dimension_semantics=("arbitrary",)),
    )(xyz1p, xyz2t, points1, points2, w0t, b0[None, :],
      w1t, b1[None, :], g0[None, :], be0[None, :], g1[None, :], be1[None, :])

    return out


# speedup vs baseline: 36.2906x; 1.0187x over previous
import jax
import jax.numpy as jnp
from jax.experimental import pallas as pl
from jax.experimental.pallas import tpu as pltpu

_EPS = 1e-5
_NB_BLK = 512
_CHUNK = 2048


def _fp_kernel(x1_ref, x2ts_ref, p1_ref, p2_ref, w0t_ref, b0_ref,
               w1t_ref, b1_ref, g0_ref, be0_ref, g1_ref, be1_ref,
               o_ref, y1s_ref, s1_ref, q1_ref):
    t = pl.program_id(0)
    steps = pl.num_programs(0) - 1
    nb = _NB_BLK

    @pl.when(t == 0)
    def _():
        s1_ref[...] = jnp.zeros_like(s1_ref)
        q1_ref[...] = jnp.zeros_like(q1_ref)

    @pl.when(t < steps)
    def _():
        x1 = x1_ref[0]
        x2ts = x2ts_ref[0]
        dsel = jnp.dot(x1, x2ts, preferred_element_type=jnp.float32)
        x1sq = jnp.sum(x1 * x1, axis=1, keepdims=True) - 1.0

        dw = dsel
        wmat = None
        wsum = None
        for k in range(3):
            v = jnp.min(dw, axis=1, keepdims=True)
            w_k = 1.0 / jnp.maximum(v + x1sq, 1e-10)
            wsum = w_k if k == 0 else wsum + w_k
            m_k = dw == v
            wmat = jnp.where(m_k, w_k, 0.0 if k == 0 else wmat)
            if k < 2:
                dw = jnp.where(m_k, jnp.float32(jnp.inf), dw)

        interp = jnp.dot(wmat, p2_ref[0], preferred_element_type=jnp.float32)
        interp = interp * (1.0 / wsum)
        x = jnp.concatenate([p1_ref[0], interp], axis=1)
        y1 = jnp.dot(x, w0t_ref[...], preferred_element_type=jnp.float32)
        y1 = y1 + b0_ref[...]
        y1s_ref[pl.ds(t * nb, nb), :] = y1
        s1_ref[...] += jnp.sum(y1, axis=0, keepdims=True)
        q1_ref[...] += jnp.sum(y1 * y1, axis=0, keepdims=True)

    @pl.when(t == steps)
    def _():
        rows = steps * nb
        rcount = jnp.float32(rows)
        mean1 = s1_ref[...] / rcount
        var1 = q1_ref[...] / rcount - mean1 * mean1
        sc1 = g0_ref[...] / jnp.sqrt(var1 + _EPS)
        sh1 = be0_ref[...] - mean1 * sc1

        w1t = w1t_ref[...]
        b1 = b1_ref[...]
        n_chunks = rows // _CHUNK
        s2 = None
        q2 = None
        for c in range(n_chunks):
            y1c = y1s_ref[pl.ds(c * _CHUNK, _CHUNK), :]
            h = jnp.maximum(y1c * sc1 + sh1, 0.0)
            y2c = jnp.dot(h, w1t, preferred_element_type=jnp.float32) + b1
            y1s_ref[pl.ds(c * _CHUNK, _CHUNK), :] = y2c
            cs = jnp.sum(y2c, axis=0, keepdims=True)
            cq = jnp.sum(y2c * y2c, axis=0, keepdims=True)
            s2 = cs if c == 0 else s2 + cs
            q2 = cq if c == 0 else q2 + cq

        mean2 = s2 / rcount
        var2 = q2 / rcount - mean2 * mean2
        sc2 = g1_ref[...] / jnp.sqrt(var2 + _EPS)
        sh2 = be1_ref[...] - mean2 * sc2

        n_per_b = o_ref.shape[1]
        for c in range(n_chunks):
            y2c = y1s_ref[pl.ds(c * _CHUNK, _CHUNK), :]
            res = jnp.maximum(y2c * sc2 + sh2, 0.0)
            bb = (c * _CHUNK) // n_per_b
            rr = (c * _CHUNK) % n_per_b
            o_ref[bb, pl.ds(rr, _CHUNK), :] = res


def kernel(xyz1, xyz2, points1, points2, W0, b0, g0, be0, W1, b1, g1, be1):
    f32 = jnp.float32
    B, N, _ = xyz1.shape
    M = xyz2.shape[1]
    D1 = points1.shape[2]
    D2 = points2.shape[2]
    O1 = W0.shape[0]
    O2 = W1.shape[0]
    R = B * N

    nb = _NB_BLK
    NB = N // nb
    steps = B * NB

    ones1 = jnp.ones((B, N, 1), f32)
    xyz1p = jnp.concatenate(
        [xyz1, ones1, jnp.zeros((B, N, 4), f32)], axis=2)
    x2sq = jnp.sum(xyz2 * xyz2, axis=2, keepdims=True)
    xyz2t = jnp.concatenate(
        [-2.0 * xyz2, x2sq, jnp.zeros((B, M, 4), f32)],
        axis=2).transpose(0, 2, 1)
    w0t = W0.T
    w1t = W1.T

    def bmap(t):
        tc = jnp.minimum(t, steps - 1)
        return (tc // NB, tc % NB, 0)

    def cmap(t):
        return (jnp.minimum(t, steps - 1) // NB, 0, 0)

    out = pl.pallas_call(
        _fp_kernel,
        grid=(steps + 1,),
        in_specs=[
            pl.BlockSpec((1, nb, 8), bmap),
            pl.BlockSpec((1, 8, M), cmap),
            pl.BlockSpec((1, nb, D1), bmap),
            pl.BlockSpec((1, M, D2), cmap),
            pl.BlockSpec((D1 + D2, O1), lambda t: (0, 0)),
            pl.BlockSpec((1, O1), lambda t: (0, 0)),
            pl.BlockSpec((O1, O2), lambda t: (0, 0)),
            pl.BlockSpec((1, O2), lambda t: (0, 0)),
            pl.BlockSpec((1, O1), lambda t: (0, 0)),
            pl.BlockSpec((1, O1), lambda t: (0, 0)),
            pl.BlockSpec((1, O2), lambda t: (0, 0)),
            pl.BlockSpec((1, O2), lambda t: (0, 0)),
        ],
        out_specs=pl.BlockSpec((B, N, O2), lambda t: (0, 0, 0)),
        out_shape=jax.ShapeDtypeStruct((B, N, O2), f32),
        scratch_shapes=[
            pltpu.VMEM((R, O1), f32),
            pltpu.VMEM((1, O1), f32),
            pltpu.VMEM((1, O1), f32),
        ],
        compiler_params=pltpu.CompilerParams(
            dimension_semantics=("arbitrary",)),
    )(xyz1p, xyz2t, points1, points2, w0t, b0[None, :],
      w1t, b1[None, :], g0[None, :], be0[None, :], g1[None, :], be1[None, :])

    return out


# speedup vs baseline: 39.4928x; 1.0882x over previous
import jax
import jax.numpy as jnp
from jax.experimental import pallas as pl
from jax.experimental.pallas import tpu as pltpu

_EPS = 1e-5
_NB_BLK = 1024
_CHUNK = 2048


def _fp_kernel(x1_ref, x2ts_ref, p1_ref, p2_ref, w0t_ref, b0_ref,
               w1t_ref, b1_ref, g0_ref, be0_ref, g1_ref, be1_ref,
               o_ref, y1s_ref, s1_ref, q1_ref):
    t = pl.program_id(0)
    steps = pl.num_programs(0) - 1
    nb = _NB_BLK

    @pl.when(t == 0)
    def _():
        s1_ref[...] = jnp.zeros_like(s1_ref)
        q1_ref[...] = jnp.zeros_like(q1_ref)

    @pl.when(t < steps)
    def _():
        x1 = x1_ref[0]
        x2ts = x2ts_ref[0]
        dots = jnp.dot(x1, x2ts, preferred_element_type=jnp.float32)
        x2sq = 0.25 * jnp.sum(x2ts * x2ts, axis=0, keepdims=True)
        dsel = dots + x2sq
        x1sq = jnp.sum(x1 * x1, axis=1, keepdims=True)

        dw = dsel
        wmat = None
        wsum = None
        for k in range(3):
            v = jnp.min(dw, axis=1, keepdims=True)
            w_k = 1.0 / jnp.maximum(v + x1sq, 1e-10)
            wsum = w_k if k == 0 else wsum + w_k
            m_k = dw == v
            wmat = jnp.where(m_k, w_k, 0.0 if k == 0 else wmat)
            if k < 2:
                dw = jnp.where(m_k, jnp.float32(jnp.inf), dw)

        interp = jnp.dot(wmat, p2_ref[0], preferred_element_type=jnp.float32)
        interp = interp * (1.0 / wsum)
        x = jnp.concatenate([p1_ref[0], interp], axis=1)
        y1 = jnp.dot(x, w0t_ref[...], preferred_element_type=jnp.float32)
        y1 = y1 + b0_ref[...]
        y1s_ref[pl.ds(t * nb, nb), :] = y1
        s1_ref[...] += jnp.sum(y1, axis=0, keepdims=True)
        q1_ref[...] += jnp.sum(y1 * y1, axis=0, keepdims=True)

    @pl.when(t == steps)
    def _():
        rows = steps * nb
        rcount = jnp.float32(rows)
        mean1 = s1_ref[...] / rcount
        var1 = q1_ref[...] / rcount - mean1 * mean1
        sc1 = g0_ref[...] / jnp.sqrt(var1 + _EPS)
        sh1 = be0_ref[...] - mean1 * sc1

        w1t = w1t_ref[...]
        b1 = b1_ref[...]
        n_chunks = rows // _CHUNK
        s2 = None
        q2 = None
        for c in range(n_chunks):
            y1c = y1s_ref[pl.ds(c * _CHUNK, _CHUNK), :]
            h = jnp.maximum(y1c * sc1 + sh1, 0.0)
            y2c = jnp.dot(h, w1t, preferred_element_type=jnp.float32) + b1
            y1s_ref[pl.ds(c * _CHUNK, _CHUNK), :] = y2c
            cs = jnp.sum(y2c, axis=0, keepdims=True)
            cq = jnp.sum(y2c * y2c, axis=0, keepdims=True)
            s2 = cs if c == 0 else s2 + cs
            q2 = cq if c == 0 else q2 + cq

        mean2 = s2 / rcount
        var2 = q2 / rcount - mean2 * mean2
        sc2 = g1_ref[...] / jnp.sqrt(var2 + _EPS)
        sh2 = be1_ref[...] - mean2 * sc2

        n_per_b = o_ref.shape[1]
        for c in range(n_chunks):
            y2c = y1s_ref[pl.ds(c * _CHUNK, _CHUNK), :]
            res = jnp.maximum(y2c * sc2 + sh2, 0.0)
            bb = (c * _CHUNK) // n_per_b
            rr = (c * _CHUNK) % n_per_b
            o_ref[bb, pl.ds(rr, _CHUNK), :] = res


def kernel(xyz1, xyz2, points1, points2, W0, b0, g0, be0, W1, b1, g1, be1):
    f32 = jnp.float32
    B, N, _ = xyz1.shape
    M = xyz2.shape[1]
    D1 = points1.shape[2]
    D2 = points2.shape[2]
    O1 = W0.shape[0]
    O2 = W1.shape[0]
    R = B * N

    nb = _NB_BLK
    NB = N // nb
    steps = B * NB

    xyz1p = jnp.pad(xyz1, ((0, 0), (0, 0), (0, 5)))
    xyz2t = (-2.0 * jnp.pad(xyz2, ((0, 0), (0, 0), (0, 5)))).transpose(0, 2, 1)
    w0t = W0.T
    w1t = W1.T

    def bmap(t):
        tc = jnp.minimum(t, steps - 1)
        return (tc // NB, tc % NB, 0)

    def cmap(t):
        return (jnp.minimum(t, steps - 1) // NB, 0, 0)

    out = pl.pallas_call(
        _fp_kernel,
        grid=(steps + 1,),
        in_specs=[
            pl.BlockSpec((1, nb, 8), bmap),
            pl.BlockSpec((1, 8, M), cmap),
            pl.BlockSpec((1, nb, D1), bmap),
            pl.BlockSpec((1, M, D2), cmap),
            pl.BlockSpec((D1 + D2, O1), lambda t: (0, 0)),
            pl.BlockSpec((1, O1), lambda t: (0, 0)),
            pl.BlockSpec((O1, O2), lambda t: (0, 0)),
            pl.BlockSpec((1, O2), lambda t: (0, 0)),
            pl.BlockSpec((1, O1), lambda t: (0, 0)),
            pl.BlockSpec((1, O1), lambda t: (0, 0)),
            pl.BlockSpec((1, O2), lambda t: (0, 0)),
            pl.BlockSpec((1, O2), lambda t: (0, 0)),
        ],
        out_specs=pl.BlockSpec((B, N, O2), lambda t: (0, 0, 0)),
        out_shape=jax.ShapeDtypeStruct((B, N, O2), f32),
        scratch_shapes=[
            pltpu.VMEM((R, O1), f32),
            pltpu.VMEM((1, O1), f32),
            pltpu.VMEM((1, O1), f32),
        ],
        compiler_params=pltpu.CompilerParams(
            dimension_semantics=("arbitrary",)),
    )(xyz1p, xyz2t, points1, points2, w0t, b0[None, :],
      w1t, b1[None, :], g0[None, :], be0[None, :], g1[None, :], be1[None, :])

    return out


# speedup vs baseline: 39.5179x; 1.0006x over previous
import jax
import jax.numpy as jnp
from jax.experimental import pallas as pl
from jax.experimental.pallas import tpu as pltpu

_EPS = 1e-5
_NB_BLK = 1024
_CHUNK = 2048


def _fp_kernel(x1_ref, x2ts_ref, p1_ref, p2_ref, w0t_ref, b0_ref,
               w1t_ref, b1_ref, g0_ref, be0_ref, g1_ref, be1_ref,
               o_ref, y1s_ref, s1_ref, q1_ref):
    t = pl.program_id(0)
    steps = pl.num_programs(0) - 1
    nb = _NB_BLK

    @pl.when(t == 0)
    def _():
        s1_ref[...] = jnp.zeros_like(s1_ref)
        q1_ref[...] = jnp.zeros_like(q1_ref)

    @pl.when(t < steps)
    def _():
        x1 = x1_ref[0]
        x2ts = x2ts_ref[0]
        dots = jnp.dot(x1, x2ts, preferred_element_type=jnp.float32)
        x2sq = 0.25 * jnp.sum(x2ts * x2ts, axis=0, keepdims=True)
        dsel = dots + x2sq
        x1sq = jnp.sum(x1 * x1, axis=1, keepdims=True)

        dw = dsel
        wmat = None
        wsum = None
        for k in range(3):
            v = jnp.min(dw, axis=1, keepdims=True)
            w_k = 1.0 / jnp.maximum(v + x1sq, 1e-10)
            wsum = w_k if k == 0 else wsum + w_k
            m_k = dw == v
            wmat = jnp.where(m_k, w_k, 0.0 if k == 0 else wmat)
            if k < 2:
                dw = jnp.where(m_k, jnp.float32(jnp.inf), dw)

        interp = jnp.dot(wmat, p2_ref[0], preferred_element_type=jnp.float32)
        interp = interp * (1.0 / wsum)
        x = jnp.concatenate([p1_ref[0], interp], axis=1)
        y1 = jax.lax.dot_general(
            x, w0t_ref[...], (((1,), (1,)), ((), ())),
            preferred_element_type=jnp.float32)
        y1 = y1 + b0_ref[...]
        y1s_ref[pl.ds(t * nb, nb), :] = y1
        s1_ref[...] += jnp.sum(y1, axis=0, keepdims=True)
        q1_ref[...] += jnp.sum(y1 * y1, axis=0, keepdims=True)

    @pl.when(t == steps)
    def _():
        rows = steps * nb
        rcount = jnp.float32(rows)
        mean1 = s1_ref[...] / rcount
        var1 = q1_ref[...] / rcount - mean1 * mean1
        sc1 = g0_ref[...] / jnp.sqrt(var1 + _EPS)
        sh1 = be0_ref[...] - mean1 * sc1

        w1t = w1t_ref[...]
        b1 = b1_ref[...]
        n_chunks = rows // _CHUNK
        s2 = None
        q2 = None
        for c in range(n_chunks):
            y1c = y1s_ref[pl.ds(c * _CHUNK, _CHUNK), :]
            h = jnp.maximum(y1c * sc1 + sh1, 0.0)
            y2c = jax.lax.dot_general(
                h, w1t, (((1,), (1,)), ((), ())),
                preferred_element_type=jnp.float32) + b1
            y1s_ref[pl.ds(c * _CHUNK, _CHUNK), :] = y2c
            cs = jnp.sum(y2c, axis=0, keepdims=True)
            cq = jnp.sum(y2c * y2c, axis=0, keepdims=True)
            s2 = cs if c == 0 else s2 + cs
            q2 = cq if c == 0 else q2 + cq

        mean2 = s2 / rcount
        var2 = q2 / rcount - mean2 * mean2
        sc2 = g1_ref[...] / jnp.sqrt(var2 + _EPS)
        sh2 = be1_ref[...] - mean2 * sc2

        n_per_b = o_ref.shape[1]
        for c in range(n_chunks):
            y2c = y1s_ref[pl.ds(c * _CHUNK, _CHUNK), :]
            res = jnp.maximum(y2c * sc2 + sh2, 0.0)
            bb = (c * _CHUNK) // n_per_b
            rr = (c * _CHUNK) % n_per_b
            o_ref[bb, pl.ds(rr, _CHUNK), :] = res


def kernel(xyz1, xyz2, points1, points2, W0, b0, g0, be0, W1, b1, g1, be1):
    f32 = jnp.float32
    B, N, _ = xyz1.shape
    M = xyz2.shape[1]
    D1 = points1.shape[2]
    D2 = points2.shape[2]
    O1 = W0.shape[0]
    O2 = W1.shape[0]
    R = B * N

    nb = _NB_BLK
    NB = N // nb
    steps = B * NB

    xyz1p = jnp.pad(xyz1, ((0, 0), (0, 0), (0, 5)))
    xyz2t = (-2.0 * jnp.pad(xyz2, ((0, 0), (0, 0), (0, 5)))).transpose(0, 2, 1)

    def bmap(t):
        tc = jnp.minimum(t, steps - 1)
        return (tc // NB, tc % NB, 0)

    def cmap(t):
        return (jnp.minimum(t, steps - 1) // NB, 0, 0)

    out = pl.pallas_call(
        _fp_kernel,
        grid=(steps + 1,),
        in_specs=[
            pl.BlockSpec((1, nb, 8), bmap),
            pl.BlockSpec((1, 8, M), cmap),
            pl.BlockSpec((1, nb, D1), bmap),
            pl.BlockSpec((1, M, D2), cmap),
            pl.BlockSpec((O1, D1 + D2), lambda t: (0, 0)),
            pl.BlockSpec((1, O1), lambda t: (0, 0)),
            pl.BlockSpec((O2, O1), lambda t: (0, 0)),
            pl.BlockSpec((1, O2), lambda t: (0, 0)),
            pl.BlockSpec((1, O1), lambda t: (0, 0)),
            pl.BlockSpec((1, O1), lambda t: (0, 0)),
            pl.BlockSpec((1, O2), lambda t: (0, 0)),
            pl.BlockSpec((1, O2), lambda t: (0, 0)),
        ],
        out_specs=pl.BlockSpec((B, N, O2), lambda t: (0, 0, 0)),
        out_shape=jax.ShapeDtypeStruct((B, N, O2), f32),
        scratch_shapes=[
            pltpu.VMEM((R, O1), f32),
            pltpu.VMEM((1, O1), f32),
            pltpu.VMEM((1, O1), f32),
        ],
        compiler_params=pltpu.CompilerParams(
            dimension_semantics=("arbitrary",)),
    )(xyz1p, xyz2t, points1, points2, W0, b0[None, :],
      W1, b1[None, :], g0[None, :], be0[None, :], g1[None, :], be1[None, :])

    return out
